# Initial kernel scaffold; baseline (speedup 1.0000x reference)
import jax
import jax.numpy as jnp
from jax.experimental import pallas as pl


def kernel(x, projections, conn, memory):
    raise NotImplementedError("write your pallas kernel here")


# speedup vs baseline: 54.2569x; 54.2569x over previous
import functools

import jax
import jax.numpy as jnp
from jax import lax
from jax.experimental import pallas as pl
from jax.experimental.pallas import tpu as pltpu
from jax.experimental.pallas import tpu_sc as plsc

R = 16
N = 2048
B = 4096
X = 8192
NB = 8
L = 16
HALF = N // 2
NG = 8
GSZ = HALF // NG
CH = GSZ // L

_mesh = plsc.VectorSubcoreMesh(core_axis_name="c", subcore_axis_name="s")


@functools.partial(
    pl.kernel,
    out_type=jax.ShapeDtypeStruct((N,), jnp.int32),
    mesh=_mesh,
    compiler_params=pltpu.CompilerParams(
        needs_layout_passes=False, use_tc_tiling_on_sc=False),
    scratch_types=[
        pltpu.VMEM((X,), jnp.int32),
        pltpu.VMEM((B,), jnp.int32),
        pltpu.VMEM((NB, HALF), jnp.int32),
        pltpu.VMEM((NG, GSZ), jnp.int32),
        pltpu.VMEM((HALF,), jnp.int32),
        pltpu.VMEM((NG, GSZ, L), jnp.float32),
        pltpu.VMEM((16, 64), jnp.int32),
        pltpu.VMEM((16,), jnp.int32),
        pltpu.VMEM((16, 64), jnp.int32),
        pltpu.VMEM((64,), jnp.int32),
        pltpu.VMEM_SHARED((16, 64), jnp.int32),
        pltpu.SemaphoreType.DMA,
    ],
)
def _ensemble_ram_sc(x_hbm, proj_hbm, conn_hbm, mem_hbm, out_hbm,
                     x_v, proj_v, conn_v, idx_v, addr_v, rows_v, bits_v,
                     rowidx_v, red_v, outst_v, shared, sem):
    cid = lax.axis_index("c")
    sid = lax.axis_index("s")
    lane = lax.iota(jnp.int32, L)

    pltpu.sync_copy(x_hbm, x_v)
    pltpu.sync_copy(proj_hbm.at[sid], proj_v)
    pltpu.sync_copy(conn_hbm.at[sid, cid], conn_v)

    nrow_base = (sid * N + cid * HALF) * 16

    def addr_body(g, _):
        for i in range(CH):
            nb = g * GSZ + i * L
            addr = jnp.zeros((L,), jnp.int32)
            for b in range(NB):
                c = conn_v[b, pl.ds(nb, L)]
                w = plsc.load_gather(proj_v, [c])
                bit = plsc.load_gather(x_v, [w])
                addr = addr + bit * (1 << b)
            addr_v[pl.ds(nb, L)] = addr
            blk = nrow_base + (nb + lane) * 16 + jnp.right_shift(addr, 4)
            idx_v[g, pl.ds(i * L, L)] = blk
        return _

    lax.fori_loop(0, NG, addr_body, None)

    copies = [
        pltpu.async_copy(mem_hbm.at[idx_v.at[g]], rows_v.at[g], sem)
        for g in range(NG)
    ]
    for c in copies:
        c.wait()

    def sel_body(g, _):
        gvec = jnp.full((L,), 0, jnp.int32) + g
        for i in range(CH):
            nb = g * GSZ + i * L
            addr = addr_v[pl.ds(nb, L)]
            rowvec = i * L + lane
            lanevec = jnp.bitwise_and(addr, 15)
            val = plsc.load_gather(rows_v, [gvec, rowvec, lanevec])
            bit = jnp.where(val > 0.5, 1, 0).astype(jnp.int32)
            bits_v[g * 2 + i // 4, pl.ds((i % 4) * L, L)] = bit
        return _

    lax.fori_loop(0, NG, sel_body, None)

    rowidx_v[...] = lane

    @pl.when(sid == 0)
    def _():
        pltpu.sync_copy(bits_v, shared)

    plsc.subcore_barrier()

    @pl.when(sid != 0)
    def _():
        pltpu.sync_copy(bits_v, shared.at[rowidx_v], add=True)

    plsc.subcore_barrier()
    pltpu.sync_copy(shared, red_v)
    W = HALF // R
    for gg in range(W // L):
        acc = red_v[sid, pl.ds(gg * L, L)]
        outst_v[pl.ds(gg * L, L)] = jnp.where(acc > R // 2, 1, 0).astype(jnp.int32)
    pltpu.sync_copy(outst_v, out_hbm.at[pl.ds(cid * HALF + sid * W, W)])


def kernel(x, projections, conn, memory):
    conn_r = conn.reshape(R, 2, HALF, NB).transpose(0, 1, 3, 2)
    mem16 = memory.reshape(R * N * 16, 16)
    out = _ensemble_ram_sc(x, projections, conn_r, mem16)
    return out.astype(jnp.uint8)
